# baseline (speedup 1.0000x reference)
import jax
import jax.numpy as jnp

_HI = jax.lax.Precision.HIGHEST
from jax import lax
from jax.experimental import pallas as pl
from jax.experimental.pallas import tpu as pltpu
from jax.experimental.pallas import tpu_sc as plsc

N = 10000
E = 320000
F_IN = 128
H1N = 8
C1 = 16
F1 = H1N * C1
C2 = 16
NCLS = 40

NC, NS = 2, 16
NT = NC * NS
EPT = E // NT
CH = 80
NCHUNK = EPT // CH
NPAD = 10240
RPT = NPAD // NS
ZR = 128

RB = 2000
GR = N // RB

_f32 = jnp.float32


def _dense1_body(x_ref, w1_ref, bds_ref, bdd_ref, mcs_ref, mcd_ref,
                 h1_ref, asc_ref, adc_ref, sm_ref, sd_ref):
    h1 = jnp.dot(x_ref[...], w1_ref[...])
    asb = jnp.dot(h1, bds_ref[...], precision=_HI)
    adb = jnp.dot(h1, bdd_ref[...], precision=_HI)
    asc = jnp.dot(h1, mcs_ref[...], precision=_HI)
    adc = jnp.dot(h1, mcd_ref[...], precision=_HI)
    h1_ref[...] = h1
    asc_ref[...] = asc
    adc_ref[...] = adc
    tb = asb + adb
    wb = jnp.exp(jnp.maximum(tb, 0.2 * tb))
    sm_ref[...] = wb * h1
    tc = asc + adc
    sd_ref[...] = jnp.exp(jnp.maximum(tc, 0.2 * tc))


def _dense1(x, w1, bds, bdd, mcs, mcd):
    full = lambda s: pl.BlockSpec(s, lambda i: (0, 0))
    rows = lambda d: pl.BlockSpec((RB, d), lambda i: (i, 0))
    return pl.pallas_call(
        _dense1_body,
        grid=(GR,),
        in_specs=[rows(F_IN), full((F_IN, F1)), full((F1, F1)), full((F1, F1)),
                  full((F1, C1)), full((F1, C1))],
        out_specs=[rows(F1), rows(C1), rows(C1), rows(F1), rows(C1)],
        out_shape=[jax.ShapeDtypeStruct((N, F1), _f32),
                   jax.ShapeDtypeStruct((N, C1), _f32),
                   jax.ShapeDtypeStruct((N, C1), _f32),
                   jax.ShapeDtypeStruct((N, F1), _f32),
                   jax.ShapeDtypeStruct((N, C1), _f32)],
    )(x, w1, bds, bdd, mcs, mcd)


def _sc1_body(src_hbm, dst_hbm, h1_hbm, asc_hbm, adc_hbm,
              pm_hbm, pd_hbm,
              accm, accd, sidx0, sidx1, didx0, didx1, hbuf0, hbuf1,
              asc0, asc1, adc0, adc1, dstg0, dstg1,
              gsem0, gsem1, ssem0, ssem1):
    c = lax.axis_index("c")
    s = lax.axis_index("s")
    sidx = [sidx0, sidx1]
    didx = [didx0, didx1]
    hbuf = [hbuf0, hbuf1]
    asc = [asc0, asc1]
    adc = [adc0, adc1]
    dstg = [dstg0, dstg1]
    gsem = [gsem0, gsem1]
    ssem = [ssem0, ssem1]

    def zrow(i, carry):
        for j in range(F1 // 16):
            hbuf0[i, pl.ds(j * 16, 16)] = jnp.zeros((16,), _f32)
        dstg0[i, :] = jnp.zeros((16,), _f32)
        return carry

    lax.fori_loop(0, CH, zrow, 0)
    for k in range(RPT // CH):
        pltpu.sync_copy(hbuf0, accm.at[pl.ds(s * RPT + k * CH, CH)])
        pltpu.sync_copy(dstg0, accd.at[pl.ds(s * RPT + k * CH, CH)])
    plsc.subcore_barrier()

    base = (c * NS + s) * EPT

    def gather_issue(p, i):
        cb = base + i * CH
        pltpu.sync_copy(src_hbm.at[pl.ds(cb, CH)], sidx[p])
        pltpu.sync_copy(dst_hbm.at[pl.ds(cb, CH)], didx[p])
        pltpu.async_copy(h1_hbm.at[sidx[p]], hbuf[p], gsem[p])
        pltpu.async_copy(asc_hbm.at[sidx[p]], asc[p], gsem[p])
        pltpu.async_copy(adc_hbm.at[didx[p]], adc[p], gsem[p])

    def gather_wait(p):
        pltpu.make_async_copy(h1_hbm.at[sidx[p]], hbuf[p], gsem[p]).wait()
        pltpu.make_async_copy(asc_hbm.at[sidx[p]], asc[p], gsem[p]).wait()
        pltpu.make_async_copy(adc_hbm.at[didx[p]], adc[p], gsem[p]).wait()

    def scatter_issue(p):
        pltpu.async_copy(hbuf[p], accm.at[didx[p]], ssem[p], add=True)
        pltpu.async_copy(dstg[p], accd.at[didx[p]], ssem[p], add=True)

    def scatter_wait(p):
        pltpu.make_async_copy(hbuf[p], accm.at[didx[p]], ssem[p]).wait()
        pltpu.make_async_copy(dstg[p], accd.at[didx[p]], ssem[p]).wait()

    hvecs = [jnp.full((16,), hh, jnp.int32) for hh in range(H1N)]

    gather_issue(0, 0)

    def gloop(g, carry):
        for b in range(2):
            i = 2 * g + b
            p, q = b, 1 - b

            @pl.when(i < NCHUNK)
            def _():
                gather_wait(p)

                @pl.when(i + 1 < NCHUNK)
                def _():
                    @pl.when(i >= 1)
                    def _():
                        scatter_wait(q)
                    gather_issue(q, i + 1)

                def edge(e, ec):
                    t = asc[p][e, :] + adc[p][e, :]
                    w16 = jnp.exp(jnp.maximum(t, 0.2 * t))
                    dstg[p][e, :] = w16
                    for hh in range(H1N):
                        sl = pl.ds(hh * 16, 16)
                        wb = w16[hvecs[hh]]
                        hbuf[p][e, sl] = wb * hbuf[p][e, sl]
                    return ec

                lax.fori_loop(0, CH, edge, 0, unroll=4)
                scatter_issue(p)
        return carry

    lax.fori_loop(0, (NCHUNK + 1) // 2, gloop, 0)
    scatter_wait(1 - (NCHUNK - 1) % 2)
    scatter_wait((NCHUNK - 1) % 2)
    plsc.subcore_barrier()
    pltpu.sync_copy(accm.at[pl.ds(s * RPT, RPT)], pm_hbm.at[c, pl.ds(s * RPT, RPT)])
    pltpu.sync_copy(accd.at[pl.ds(s * RPT, RPT)], pd_hbm.at[c, pl.ds(s * RPT, RPT)])


def _sc_layer1(src, dst, h1, asc, adc):
    mesh = plsc.VectorSubcoreMesh(core_axis_name="c", subcore_axis_name="s",
                                  num_cores=NC, num_subcores=NS)
    f = pl.kernel(
        _sc1_body,
        out_type=[jax.ShapeDtypeStruct((NC, NPAD, F1), _f32),
                  jax.ShapeDtypeStruct((NC, NPAD, C1), _f32)],
        mesh=mesh,
        compiler_params=pltpu.CompilerParams(use_tc_tiling_on_sc=False),
        scratch_types=(
            [pltpu.VMEM_SHARED((NPAD, F1), _f32),
             pltpu.VMEM_SHARED((NPAD, C1), _f32)]
            + [pltpu.VMEM((CH,), jnp.int32)] * 4
            + [pltpu.VMEM((CH, F1), _f32)] * 2
            + [pltpu.VMEM((CH, C1), _f32)] * 6
            + [pltpu.SemaphoreType.DMA] * 4
        ),
    )
    return f(src, dst, h1, asc, adc)


def _dense2_body(pm0_ref, pm1_ref, pd0_ref, pd1_ref, sm_ref, sd_ref,
                 r8_ref, w2_ref, as2m_ref, ad2m_ref, b1_ref,
                 h2_ref, as2_ref, ad2_ref, s2m_ref, s2d_ref):
    num = pm0_ref[...] + pm1_ref[...] + sm_ref[...]
    den16 = pd0_ref[...] + pd1_ref[...] + sd_ref[...]
    denb = jnp.dot(den16, r8_ref[...], precision=_HI)
    out1 = num / denb + b1_ref[...]
    helu = jnp.where(out1 > 0, out1, jnp.exp(out1) - 1.0)
    h2 = jnp.dot(helu, w2_ref[...])
    as2 = jnp.dot(h2, as2m_ref[...], precision=_HI)
    ad2 = jnp.dot(h2, ad2m_ref[...], precision=_HI)
    h2_ref[...] = h2
    as2_ref[...] = as2
    ad2_ref[...] = ad2
    t = as2 + ad2
    w = jnp.exp(jnp.maximum(t, 0.2 * t))
    s2m_ref[...] = w * h2
    s2d_ref[...] = w


def _dense2(pm0, pm1, pd0, pd1, sm, sd, r8, w2, as2m, ad2m, b1):
    full = lambda s: pl.BlockSpec(s, lambda i: (0, 0))
    rows = lambda d: pl.BlockSpec((RB, d), lambda i: (i, 0))
    return pl.pallas_call(
        _dense2_body,
        grid=(GR,),
        in_specs=[rows(F1), rows(F1), rows(C1), rows(C1), rows(F1), rows(C1),
                  full((C1, F1)), full((F1, C2)), full((C2, C2)),
                  full((C2, C2)), full((1, F1))],
        out_specs=[rows(C2), rows(C2), rows(C2), rows(C2), rows(C2)],
        out_shape=[jax.ShapeDtypeStruct((N, C2), _f32) for _ in range(5)],
    )(pm0, pm1, pd0, pd1, sm, sd, r8, w2, as2m, ad2m, b1)


def _sc2_body(src_hbm, dst_hbm, h2_hbm, as2_hbm, ad2_hbm,
              pm_hbm, pd_hbm,
              accm, accd, sidx0, sidx1, didx0, didx1, hbuf0, hbuf1,
              as20, as21, ad20, ad21, dstg0, dstg1,
              gsem0, gsem1, ssem0, ssem1):
    c = lax.axis_index("c")
    s = lax.axis_index("s")
    sidx = [sidx0, sidx1]
    didx = [didx0, didx1]
    hbuf = [hbuf0, hbuf1]
    as2 = [as20, as21]
    ad2 = [ad20, ad21]
    dstg = [dstg0, dstg1]
    gsem = [gsem0, gsem1]
    ssem = [ssem0, ssem1]

    def zrow(i, carry):
        dstg0[i, :] = jnp.zeros((16,), _f32)
        return carry

    lax.fori_loop(0, CH, zrow, 0)
    for k in range(RPT // CH):
        pltpu.sync_copy(dstg0, accm.at[pl.ds(s * RPT + k * CH, CH)])
        pltpu.sync_copy(dstg0, accd.at[pl.ds(s * RPT + k * CH, CH)])
    plsc.subcore_barrier()

    base = (c * NS + s) * EPT

    def gather_issue(p, i):
        cb = base + i * CH
        pltpu.sync_copy(src_hbm.at[pl.ds(cb, CH)], sidx[p])
        pltpu.sync_copy(dst_hbm.at[pl.ds(cb, CH)], didx[p])
        pltpu.async_copy(h2_hbm.at[sidx[p]], hbuf[p], gsem[p])
        pltpu.async_copy(as2_hbm.at[sidx[p]], as2[p], gsem[p])
        pltpu.async_copy(ad2_hbm.at[didx[p]], ad2[p], gsem[p])

    def gather_wait(p):
        pltpu.make_async_copy(h2_hbm.at[sidx[p]], hbuf[p], gsem[p]).wait()
        pltpu.make_async_copy(as2_hbm.at[sidx[p]], as2[p], gsem[p]).wait()
        pltpu.make_async_copy(ad2_hbm.at[didx[p]], ad2[p], gsem[p]).wait()

    def scatter_issue(p):
        pltpu.async_copy(hbuf[p], accm.at[didx[p]], ssem[p], add=True)
        pltpu.async_copy(dstg[p], accd.at[didx[p]], ssem[p], add=True)

    def scatter_wait(p):
        pltpu.make_async_copy(hbuf[p], accm.at[didx[p]], ssem[p]).wait()
        pltpu.make_async_copy(dstg[p], accd.at[didx[p]], ssem[p]).wait()

    gather_issue(0, 0)

    def gloop(g, carry):
        for b in range(2):
            i = 2 * g + b
            p, q = b, 1 - b

            @pl.when(i < NCHUNK)
            def _():
                gather_wait(p)

                @pl.when(i + 1 < NCHUNK)
                def _():
                    @pl.when(i >= 1)
                    def _():
                        scatter_wait(q)
                    gather_issue(q, i + 1)

                def edge(e, ec):
                    t = as2[p][e, :] + ad2[p][e, :]
                    w = jnp.exp(jnp.maximum(t, 0.2 * t))
                    hbuf[p][e, :] = w * hbuf[p][e, :]
                    dstg[p][e, :] = w
                    return ec

                lax.fori_loop(0, CH, edge, 0, unroll=4)
                scatter_issue(p)
        return carry

    lax.fori_loop(0, (NCHUNK + 1) // 2, gloop, 0)
    scatter_wait(1 - (NCHUNK - 1) % 2)
    scatter_wait((NCHUNK - 1) % 2)
    plsc.subcore_barrier()
    pltpu.sync_copy(accm.at[pl.ds(s * RPT, RPT)], pm_hbm.at[c, pl.ds(s * RPT, RPT)])
    pltpu.sync_copy(accd.at[pl.ds(s * RPT, RPT)], pd_hbm.at[c, pl.ds(s * RPT, RPT)])


def _sc_layer2(src, dst, h2, as2, ad2):
    mesh = plsc.VectorSubcoreMesh(core_axis_name="c", subcore_axis_name="s",
                                  num_cores=NC, num_subcores=NS)
    f = pl.kernel(
        _sc2_body,
        out_type=[jax.ShapeDtypeStruct((NC, NPAD, C2), _f32),
                  jax.ShapeDtypeStruct((NC, NPAD, C2), _f32)],
        mesh=mesh,
        compiler_params=pltpu.CompilerParams(use_tc_tiling_on_sc=False),
        scratch_types=(
            [pltpu.VMEM_SHARED((NPAD, C2), _f32),
             pltpu.VMEM_SHARED((NPAD, C2), _f32)]
            + [pltpu.VMEM((CH,), jnp.int32)] * 4
            + [pltpu.VMEM((CH, C2), _f32)] * 8
            + [pltpu.SemaphoreType.DMA] * 4
        ),
    )
    return f(src, dst, h2, as2, ad2)


def _dense3_body(pm0_ref, pm1_ref, pd0_ref, pd1_ref, s2m_ref, s2d_ref,
                 b2_ref, wc_ref, bc_ref,
                 logits_ref, emb_ref, soft_ref, hard_ref):
    num = pm0_ref[...] + pm1_ref[...] + s2m_ref[...]
    den = pd0_ref[...] + pd1_ref[...] + s2d_ref[...]
    emb = num / den + b2_ref[...]
    logits = jnp.dot(emb, wc_ref[...]) + bc_ref[...]
    m = jnp.max(logits, axis=1, keepdims=True)
    ex = jnp.exp(logits - m)
    soft = ex / jnp.sum(ex, axis=1, keepdims=True)
    idx = lax.broadcasted_iota(jnp.int32, logits.shape, 1)
    hard = jnp.min(jnp.where(logits >= m, idx, NCLS), axis=1)
    emb_ref[...] = emb
    logits_ref[...] = logits
    soft_ref[...] = soft
    hard_ref[...] = hard[:, None]


def _dense3(pm0, pm1, pd0, pd1, s2m, s2d, b2, wc, bc):
    full = lambda s: pl.BlockSpec(s, lambda i: (0, 0))
    rows = lambda d: pl.BlockSpec((RB, d), lambda i: (i, 0))
    return pl.pallas_call(
        _dense3_body,
        grid=(GR,),
        in_specs=[rows(C2), rows(C2), rows(C2), rows(C2), rows(C2), rows(C2),
                  full((1, C2)), full((C2, NCLS)), full((1, NCLS))],
        out_specs=[rows(NCLS), rows(C2), rows(NCLS), rows(1)],
        out_shape=[jax.ShapeDtypeStruct((N, NCLS), _f32),
                   jax.ShapeDtypeStruct((N, C2), _f32),
                   jax.ShapeDtypeStruct((N, NCLS), _f32),
                   jax.ShapeDtypeStruct((N, 1), jnp.int32)],
    )(pm0, pm1, pd0, pd1, s2m, s2d, b2, wc, bc)


def kernel(x, edge_index, W1, att_src1, att_dst1, b1, W2, att_src2, att_dst2,
           b2, Wc, bc):
    src = edge_index[0]
    dst = edge_index[1]

    eye8 = jnp.eye(H1N, dtype=_f32)
    ones_c = jnp.ones((C1,), _f32)
    bds = (att_src1[:, :, None, None] * eye8[:, None, :, None]
           * ones_c[None, None, None, :]).reshape(F1, F1)
    bdd = (att_dst1[:, :, None, None] * eye8[:, None, :, None]
           * ones_c[None, None, None, :]).reshape(F1, F1)
    mcs = jnp.concatenate(
        [(att_src1[:, :, None] * eye8[:, None, :]).reshape(F1, H1N),
         jnp.zeros((F1, C1 - H1N), _f32)], axis=1)
    mcd = jnp.concatenate(
        [(att_dst1[:, :, None] * eye8[:, None, :]).reshape(F1, H1N),
         jnp.zeros((F1, C1 - H1N), _f32)], axis=1)
    r8 = (jnp.eye(C1, H1N, dtype=_f32)[:, :, None]
          * ones_c[None, None, :]).reshape(C1, F1)
    as2m = att_src2[0][:, None] * jnp.ones((1, C2), _f32)
    ad2m = att_dst2[0][:, None] * jnp.ones((1, C2), _f32)

    h1, asc, adc, sm, sd = _dense1(x, W1, bds, bdd, mcs, mcd)
    pm, pd = _sc_layer1(src, dst, h1, asc, adc)
    h2, as2, ad2, s2m, s2d = _dense2(pm[0, :N], pm[1, :N], pd[0, :N], pd[1, :N], sm, sd,
                                     r8, W2, as2m, ad2m, b1[None, :])
    p2m, p2d = _sc_layer2(src, dst, h2, as2, ad2)
    logits, emb, soft, hard2d = _dense3(p2m[0, :N], p2m[1, :N], p2d[0, :N], p2d[1, :N],
                                        s2m, s2d, b2[None, :], Wc, bc[None, :])
    return (logits, emb, soft, hard2d.reshape(N))


# speedup vs baseline: 1.2936x; 1.2936x over previous
import jax
import jax.numpy as jnp

_HI = jax.lax.Precision.HIGHEST
from jax import lax
from jax.experimental import pallas as pl
from jax.experimental.pallas import tpu as pltpu
from jax.experimental.pallas import tpu_sc as plsc

N = 10000
E = 320000
F_IN = 128
H1N = 8
C1 = 16
F1 = H1N * C1
C2 = 16
NCLS = 40

NC, NS = 2, 16
NT = NC * NS
EPT = E // NT
CH = 80
NCHUNK = EPT // CH
SUP = 5
NSUP = EPT // (SUP * CH)
NPAD = 10240
RPT = NPAD // NS
ZR = 128

RB = 2000
GR = N // RB

_f32 = jnp.float32


def _dense1_body(x_ref, w1_ref, bds_ref, bdd_ref, mcs_ref, mcd_ref,
                 h1_ref, asc_ref, adc_ref, sm_ref, sd_ref):
    h1 = jnp.dot(x_ref[...], w1_ref[...])
    asb = jnp.dot(h1, bds_ref[...], precision=_HI)
    adb = jnp.dot(h1, bdd_ref[...], precision=_HI)
    asc = jnp.dot(h1, mcs_ref[...], precision=_HI)
    adc = jnp.dot(h1, mcd_ref[...], precision=_HI)
    h1_ref[...] = h1
    asc_ref[...] = asc
    adc_ref[...] = adc
    tb = asb + adb
    wb = jnp.exp(jnp.maximum(tb, 0.2 * tb))
    sm_ref[...] = wb * h1
    tc = asc + adc
    sd_ref[...] = jnp.exp(jnp.maximum(tc, 0.2 * tc))


def _dense1(x, w1, bds, bdd, mcs, mcd):
    full = lambda s: pl.BlockSpec(s, lambda i: (0, 0))
    rows = lambda d: pl.BlockSpec((RB, d), lambda i: (i, 0))
    return pl.pallas_call(
        _dense1_body,
        grid=(GR,),
        in_specs=[rows(F_IN), full((F_IN, F1)), full((F1, F1)), full((F1, F1)),
                  full((F1, C1)), full((F1, C1))],
        out_specs=[rows(F1), rows(C1), rows(C1), rows(F1), rows(C1)],
        out_shape=[jax.ShapeDtypeStruct((N, F1), _f32),
                   jax.ShapeDtypeStruct((N, C1), _f32),
                   jax.ShapeDtypeStruct((N, C1), _f32),
                   jax.ShapeDtypeStruct((N, F1), _f32),
                   jax.ShapeDtypeStruct((N, C1), _f32)],
    )(x, w1, bds, bdd, mcs, mcd)


def _sc1_body(src_hbm, dst_hbm, h1_hbm, asc_hbm, adc_hbm,
              pm_hbm, pd_hbm,
              accm, accd, sidx0, sidx1, didx0, didx1, hbuf0, hbuf1,
              asc0, asc1, adc0, adc1, dstg0, dstg1,
              gsem0, gsem1, ssem0, ssem1):
    c = lax.axis_index("c")
    s = lax.axis_index("s")
    sidx = [sidx0, sidx1]
    didx = [didx0, didx1]
    hbuf = [hbuf0, hbuf1]
    asc = [asc0, asc1]
    adc = [adc0, adc1]
    dstg = [dstg0, dstg1]
    gsem = [gsem0, gsem1]
    ssem = [ssem0, ssem1]

    def zrow(i, carry):
        for j in range(F1 // 16):
            hbuf0[i, pl.ds(j * 16, 16)] = jnp.zeros((16,), _f32)
        dstg0[i, :] = jnp.zeros((16,), _f32)
        return carry

    lax.fori_loop(0, CH, zrow, 0)
    for k in range(RPT // CH):
        pltpu.sync_copy(hbuf0, accm.at[pl.ds(s * RPT + k * CH, CH)])
        pltpu.sync_copy(dstg0, accd.at[pl.ds(s * RPT + k * CH, CH)])
    plsc.subcore_barrier()

    base = (c * NS + s) * EPT

    def gather_issue(p, i):
        cb = base + i * CH
        pltpu.sync_copy(src_hbm.at[pl.ds(cb, CH)], sidx[p])
        pltpu.sync_copy(dst_hbm.at[pl.ds(cb, CH)], didx[p])
        pltpu.async_copy(h1_hbm.at[sidx[p]], hbuf[p], gsem[p])
        pltpu.async_copy(asc_hbm.at[sidx[p]], asc[p], gsem[p])
        pltpu.async_copy(adc_hbm.at[didx[p]], adc[p], gsem[p])

    def gather_wait(p):
        pltpu.make_async_copy(h1_hbm.at[sidx[p]], hbuf[p], gsem[p]).wait()
        pltpu.make_async_copy(asc_hbm.at[sidx[p]], asc[p], gsem[p]).wait()
        pltpu.make_async_copy(adc_hbm.at[didx[p]], adc[p], gsem[p]).wait()

    def scatter_issue(p):
        pltpu.async_copy(hbuf[p], accm.at[didx[p]], ssem[p], add=True)
        pltpu.async_copy(dstg[p], accd.at[didx[p]], ssem[p], add=True)

    def scatter_wait(p):
        pltpu.make_async_copy(hbuf[p], accm.at[didx[p]], ssem[p]).wait()
        pltpu.make_async_copy(dstg[p], accd.at[didx[p]], ssem[p]).wait()

    hvecs = [jnp.full((16,), hh, jnp.int32) for hh in range(H1N)]

    gather_issue(0, 0)

    def gloop(g, carry):
        for b in range(2):
            i = 2 * g + b
            p, q = b, 1 - b

            @pl.when(i < NCHUNK)
            def _():
                gather_wait(p)

                @pl.when(i + 1 < NCHUNK)
                def _():
                    @pl.when(i >= 1)
                    def _():
                        scatter_wait(q)
                    gather_issue(q, i + 1)

                def edge(e, ec):
                    t = asc[p][e, :] + adc[p][e, :]
                    w16 = jnp.exp(jnp.maximum(t, 0.2 * t))
                    dstg[p][e, :] = w16
                    for hh in range(H1N):
                        sl = pl.ds(hh * 16, 16)
                        wb = w16[hvecs[hh]]
                        hbuf[p][e, sl] = wb * hbuf[p][e, sl]
                    return ec

                lax.fori_loop(0, CH, edge, 0)
                scatter_issue(p)
        return carry

    lax.fori_loop(0, (NCHUNK + 1) // 2, gloop, 0)
    scatter_wait(1 - (NCHUNK - 1) % 2)
    scatter_wait((NCHUNK - 1) % 2)
    plsc.subcore_barrier()
    pltpu.sync_copy(accm.at[pl.ds(s * RPT, RPT)], pm_hbm.at[c, pl.ds(s * RPT, RPT)])
    pltpu.sync_copy(accd.at[pl.ds(s * RPT, RPT)], pd_hbm.at[c, pl.ds(s * RPT, RPT)])


def _sc_layer1(src, dst, h1, asc, adc):
    mesh = plsc.VectorSubcoreMesh(core_axis_name="c", subcore_axis_name="s",
                                  num_cores=NC, num_subcores=NS)
    f = pl.kernel(
        _sc1_body,
        out_type=[jax.ShapeDtypeStruct((NC, NPAD, F1), _f32),
                  jax.ShapeDtypeStruct((NC, NPAD, C1), _f32)],
        mesh=mesh,
        compiler_params=pltpu.CompilerParams(use_tc_tiling_on_sc=False),
        scratch_types=(
            [pltpu.VMEM_SHARED((NPAD, F1), _f32),
             pltpu.VMEM_SHARED((NPAD, C1), _f32)]
            + [pltpu.VMEM((CH,), jnp.int32)] * 4
            + [pltpu.VMEM((CH, F1), _f32)] * 2
            + [pltpu.VMEM((CH, C1), _f32)] * 6
            + [pltpu.SemaphoreType.DMA] * 4
        ),
    )
    return f(src, dst, h1, asc, adc)


def _dense2_body(pm0_ref, pm1_ref, pd0_ref, pd1_ref, sm_ref, sd_ref,
                 r8_ref, w2_ref, as2m_ref, ad2m_ref, b1_ref,
                 h2_ref, as2_ref, ad2_ref, s2m_ref, s2d_ref):
    num = pm0_ref[...] + pm1_ref[...] + sm_ref[...]
    den16 = pd0_ref[...] + pd1_ref[...] + sd_ref[...]
    denb = jnp.dot(den16, r8_ref[...], precision=_HI)
    out1 = num / denb + b1_ref[...]
    helu = jnp.where(out1 > 0, out1, jnp.exp(out1) - 1.0)
    h2 = jnp.dot(helu, w2_ref[...])
    as2 = jnp.dot(h2, as2m_ref[...], precision=_HI)
    ad2 = jnp.dot(h2, ad2m_ref[...], precision=_HI)
    h2_ref[...] = h2
    as2_ref[...] = as2
    ad2_ref[...] = ad2
    t = as2 + ad2
    w = jnp.exp(jnp.maximum(t, 0.2 * t))
    s2m_ref[...] = w * h2
    s2d_ref[...] = w


def _dense2(pm0, pm1, pd0, pd1, sm, sd, r8, w2, as2m, ad2m, b1):
    full = lambda s: pl.BlockSpec(s, lambda i: (0, 0))
    rows = lambda d: pl.BlockSpec((RB, d), lambda i: (i, 0))
    return pl.pallas_call(
        _dense2_body,
        grid=(GR,),
        in_specs=[rows(F1), rows(F1), rows(C1), rows(C1), rows(F1), rows(C1),
                  full((C1, F1)), full((F1, C2)), full((C2, C2)),
                  full((C2, C2)), full((1, F1))],
        out_specs=[rows(C2), rows(C2), rows(C2), rows(C2), rows(C2)],
        out_shape=[jax.ShapeDtypeStruct((N, C2), _f32) for _ in range(5)],
    )(pm0, pm1, pd0, pd1, sm, sd, r8, w2, as2m, ad2m, b1)


def _sc2_body(src_hbm, dst_hbm, h2_hbm, as2_hbm, ad2_hbm,
              pm_hbm, pd_hbm,
              accm, accd, sidx0, sidx1, didx0, didx1, hbuf0, hbuf1,
              as20, as21, ad20, ad21, dstg0, dstg1,
              gsem0, gsem1, ssem0, ssem1):
    c = lax.axis_index("c")
    s = lax.axis_index("s")
    sidx = [sidx0, sidx1]
    didx = [didx0, didx1]
    hbuf = [hbuf0, hbuf1]
    as2 = [as20, as21]
    ad2 = [ad20, ad21]
    dstg = [dstg0, dstg1]
    gsem = [gsem0, gsem1]
    ssem = [ssem0, ssem1]
    SCH = SUP * CH

    def zrow(i, carry):
        dstg0[i, :] = jnp.zeros((16,), _f32)
        return carry

    lax.fori_loop(0, SCH, zrow, 0)
    pltpu.sync_copy(dstg0, accm.at[pl.ds(s * RPT, SCH)])
    pltpu.sync_copy(dstg0, accd.at[pl.ds(s * RPT, SCH)])
    pltpu.sync_copy(dstg0.at[pl.ds(0, RPT - SCH)],
                    accm.at[pl.ds(s * RPT + SCH, RPT - SCH)])
    pltpu.sync_copy(dstg0.at[pl.ds(0, RPT - SCH)],
                    accd.at[pl.ds(s * RPT + SCH, RPT - SCH)])
    plsc.subcore_barrier()

    brow = (c * NS + s) * NCHUNK

    def gather_issue(p, i):
        pltpu.sync_copy(src_hbm.at[pl.ds(brow + i * SUP, SUP)], sidx[p])
        pltpu.sync_copy(dst_hbm.at[pl.ds(brow + i * SUP, SUP)], didx[p])
        for j in range(SUP):
            sl = pl.ds(j * CH, CH)
            pltpu.async_copy(h2_hbm.at[sidx[p].at[j]], hbuf[p].at[sl], gsem[p])
            pltpu.async_copy(as2_hbm.at[sidx[p].at[j]], as2[p].at[sl], gsem[p])
            pltpu.async_copy(ad2_hbm.at[didx[p].at[j]], ad2[p].at[sl], gsem[p])

    def gather_wait(p):
        for j in range(SUP):
            sl = pl.ds(j * CH, CH)
            pltpu.make_async_copy(h2_hbm.at[sidx[p].at[j]], hbuf[p].at[sl], gsem[p]).wait()
            pltpu.make_async_copy(as2_hbm.at[sidx[p].at[j]], as2[p].at[sl], gsem[p]).wait()
            pltpu.make_async_copy(ad2_hbm.at[didx[p].at[j]], ad2[p].at[sl], gsem[p]).wait()

    def scatter_issue(p):
        for j in range(SUP):
            sl = pl.ds(j * CH, CH)
            pltpu.async_copy(hbuf[p].at[sl], accm.at[didx[p].at[j]], ssem[p], add=True)
            pltpu.async_copy(dstg[p].at[sl], accd.at[didx[p].at[j]], ssem[p], add=True)

    def scatter_wait(p):
        for j in range(SUP):
            sl = pl.ds(j * CH, CH)
            pltpu.make_async_copy(hbuf[p].at[sl], accm.at[didx[p].at[j]], ssem[p]).wait()
            pltpu.make_async_copy(dstg[p].at[sl], accd.at[didx[p].at[j]], ssem[p]).wait()

    gather_issue(0, 0)

    def gloop(g, carry):
        for b in range(2):
            i = 2 * g + b
            p, q = b, 1 - b

            @pl.when(i < NSUP)
            def _():
                gather_wait(p)

                @pl.when(i + 1 < NSUP)
                def _():
                    @pl.when(i >= 1)
                    def _():
                        scatter_wait(q)
                    gather_issue(q, i + 1)

                def edge(e, ec):
                    t = as2[p][e, :] + ad2[p][e, :]
                    w = jnp.exp(jnp.maximum(t, 0.2 * t))
                    hbuf[p][e, :] = w * hbuf[p][e, :]
                    dstg[p][e, :] = w
                    return ec

                lax.fori_loop(0, SCH, edge, 0)
                scatter_issue(p)
        return carry

    lax.fori_loop(0, (NSUP + 1) // 2, gloop, 0)
    scatter_wait(1 - (NSUP - 1) % 2)
    scatter_wait((NSUP - 1) % 2)
    plsc.subcore_barrier()
    pltpu.sync_copy(accm.at[pl.ds(s * RPT, RPT)], pm_hbm.at[c, pl.ds(s * RPT, RPT)])
    pltpu.sync_copy(accd.at[pl.ds(s * RPT, RPT)], pd_hbm.at[c, pl.ds(s * RPT, RPT)])


def _sc_layer2(src2, dst2, h2, as2, ad2):
    mesh = plsc.VectorSubcoreMesh(core_axis_name="c", subcore_axis_name="s",
                                  num_cores=NC, num_subcores=NS)
    f = pl.kernel(
        _sc2_body,
        out_type=[jax.ShapeDtypeStruct((NC, NPAD, C2), _f32),
                  jax.ShapeDtypeStruct((NC, NPAD, C2), _f32)],
        mesh=mesh,
        compiler_params=pltpu.CompilerParams(use_tc_tiling_on_sc=False),
        scratch_types=(
            [pltpu.VMEM_SHARED((NPAD, C2), _f32),
             pltpu.VMEM_SHARED((NPAD, C2), _f32)]
            + [pltpu.VMEM((SUP, CH), jnp.int32)] * 4
            + [pltpu.VMEM((SUP * CH, C2), _f32)] * 8
            + [pltpu.SemaphoreType.DMA] * 4
        ),
    )
    return f(src2, dst2, h2, as2, ad2)


def _dense3_body(pm0_ref, pm1_ref, pd0_ref, pd1_ref, s2m_ref, s2d_ref,
                 b2_ref, wc_ref, bc_ref,
                 logits_ref, emb_ref, soft_ref, hard_ref):
    num = pm0_ref[...] + pm1_ref[...] + s2m_ref[...]
    den = pd0_ref[...] + pd1_ref[...] + s2d_ref[...]
    emb = num / den + b2_ref[...]
    logits = jnp.dot(emb, wc_ref[...]) + bc_ref[...]
    m = jnp.max(logits, axis=1, keepdims=True)
    ex = jnp.exp(logits - m)
    soft = ex / jnp.sum(ex, axis=1, keepdims=True)
    idx = lax.broadcasted_iota(jnp.int32, logits.shape, 1)
    hard = jnp.min(jnp.where(logits >= m, idx, NCLS), axis=1)
    emb_ref[...] = emb
    logits_ref[...] = logits
    soft_ref[...] = soft
    hard_ref[...] = hard[:, None]


def _dense3(pm0, pm1, pd0, pd1, s2m, s2d, b2, wc, bc):
    full = lambda s: pl.BlockSpec(s, lambda i: (0, 0))
    rows = lambda d: pl.BlockSpec((RB, d), lambda i: (i, 0))
    return pl.pallas_call(
        _dense3_body,
        grid=(GR,),
        in_specs=[rows(C2), rows(C2), rows(C2), rows(C2), rows(C2), rows(C2),
                  full((1, C2)), full((C2, NCLS)), full((1, NCLS))],
        out_specs=[rows(NCLS), rows(C2), rows(NCLS), rows(1)],
        out_shape=[jax.ShapeDtypeStruct((N, NCLS), _f32),
                   jax.ShapeDtypeStruct((N, C2), _f32),
                   jax.ShapeDtypeStruct((N, NCLS), _f32),
                   jax.ShapeDtypeStruct((N, 1), jnp.int32)],
    )(pm0, pm1, pd0, pd1, s2m, s2d, b2, wc, bc)


def kernel(x, edge_index, W1, att_src1, att_dst1, b1, W2, att_src2, att_dst2,
           b2, Wc, bc):
    src = edge_index[0]
    dst = edge_index[1]

    eye8 = jnp.eye(H1N, dtype=_f32)
    ones_c = jnp.ones((C1,), _f32)
    bds = (att_src1[:, :, None, None] * eye8[:, None, :, None]
           * ones_c[None, None, None, :]).reshape(F1, F1)
    bdd = (att_dst1[:, :, None, None] * eye8[:, None, :, None]
           * ones_c[None, None, None, :]).reshape(F1, F1)
    mcs = jnp.concatenate(
        [(att_src1[:, :, None] * eye8[:, None, :]).reshape(F1, H1N),
         jnp.zeros((F1, C1 - H1N), _f32)], axis=1)
    mcd = jnp.concatenate(
        [(att_dst1[:, :, None] * eye8[:, None, :]).reshape(F1, H1N),
         jnp.zeros((F1, C1 - H1N), _f32)], axis=1)
    r8 = (jnp.eye(C1, H1N, dtype=_f32)[:, :, None]
          * ones_c[None, None, :]).reshape(C1, F1)
    as2m = att_src2[0][:, None] * jnp.ones((1, C2), _f32)
    ad2m = att_dst2[0][:, None] * jnp.ones((1, C2), _f32)

    h1, asc, adc, sm, sd = _dense1(x, W1, bds, bdd, mcs, mcd)
    pm, pd = _sc_layer1(src, dst, h1, asc, adc)
    h2, as2, ad2, s2m, s2d = _dense2(pm[0, :N], pm[1, :N], pd[0, :N], pd[1, :N], sm, sd,
                                     r8, W2, as2m, ad2m, b1[None, :])
    p2m, p2d = _sc_layer2(src.reshape(E // CH, CH), dst.reshape(E // CH, CH), h2, as2, ad2)
    logits, emb, soft, hard2d = _dense3(p2m[0, :N], p2m[1, :N], p2d[0, :N], p2d[1, :N],
                                        s2m, s2d, b2[None, :], Wc, bc[None, :])
    return (logits, emb, soft, hard2d.reshape(N))
